# baseline (speedup 1.0000x reference)
import functools

import jax
import jax.numpy as jnp
from jax import lax
from jax.experimental import pallas as pl
from jax.experimental.pallas import tpu as pltpu

_TOPK_PERCENT = 0.3


def _select_weights(w_row, k):
    n = w_row.shape[1]
    wi = lax.bitcast_convert_type(w_row, jnp.int32)

    def bs_body(_, lohi):
        lo, hi = lohi
        mid = (lo + hi) // 2
        cnt = jnp.sum((wi >= mid).astype(jnp.int32))
        ge = cnt >= k
        return jnp.where(ge, mid, lo), jnp.where(ge, hi, mid)

    lo, _ = lax.fori_loop(
        0, 31, bs_body, (jnp.int32(0), jnp.int32(0x3F800001))
    )
    t = lo

    gt = wi > t
    eq = wi == t
    n_gt = jnp.sum(gt.astype(jnp.int32))
    extra = k - n_gt

    idx = lax.broadcasted_iota(jnp.int32, (1, n), 1)

    def bs2_body(_, lohi):
        lo2, hi2 = lohi
        mid = (lo2 + hi2) // 2
        cnt = jnp.sum((eq & (idx < mid)).astype(jnp.int32))
        ge = cnt >= extra
        return jnp.where(ge, lo2, mid), jnp.where(ge, mid, hi2)

    _, m = lax.fori_loop(
        0, 14, bs2_body, (jnp.int32(0), jnp.int32(n))
    )

    sel = gt | (eq & (idx < m))
    return jnp.where(sel, w_row, 0.0)


def _main_body(x_ref, wa1_ref, ba1_ref, wa2t_ref, ba2_ref, w_ref, emb_ref,
               *, k, nc=1024):
    n = x_ref.shape[1]

    for c in range(n // nc):
        xb = x_ref[0, pl.ds(c * nc, nc), :]
        h = jax.nn.gelu(
            jnp.dot(xb, wa1_ref[...], preferred_element_type=jnp.float32)
            + ba1_ref[...]
        )
        z = (
            lax.dot_general(
                wa2t_ref[...], h,
                dimension_numbers=(((1,), (1,)), ((), ())),
                preferred_element_type=jnp.float32,
            )
            + ba2_ref[...]
        )
        w_ref[0, :, pl.ds(c * nc, nc)] = jax.nn.sigmoid(z)

    w_row = w_ref[0]
    wm_row = _select_weights(w_row, k)
    emb_ref[0] = wm_row[:, 0:512]


def _proj_body(emb_ref, wp1_ref, bp1_ref, wp2_ref, bp2_ref, out_ref):
    h = jax.nn.gelu(
        jnp.dot(emb_ref[...], wp1_ref[...], preferred_element_type=jnp.float32)
        + bp1_ref[...]
    )
    out_ref[...] = (
        jnp.dot(h, wp2_ref[...], preferred_element_type=jnp.float32) + bp2_ref[...]
    )


def kernel(x, Wa1, ba1, Wa2, ba2, Wp1, bp1, Wp2, bp2):
    b, n, d = x.shape
    hdim = Wa1.shape[1]
    k = max(1, int(n * _TOPK_PERCENT))

    ba1r = ba1.reshape(1, hdim)
    wa2t = Wa2.reshape(1, hdim)
    ba2r = ba2.reshape(1, 1)
    bp1r = bp1.reshape(1, d)
    bp2r = bp2.reshape(1, d)

    weights, embs = pl.pallas_call(
        functools.partial(_main_body, k=k),
        grid=(b,),
        in_specs=[
            pl.BlockSpec((1, n, d), lambda i: (i, 0, 0)),
            pl.BlockSpec((d, hdim), lambda i: (0, 0)),
            pl.BlockSpec((1, hdim), lambda i: (0, 0)),
            pl.BlockSpec((1, hdim), lambda i: (0, 0)),
            pl.BlockSpec((1, 1), lambda i: (0, 0)),
        ],
        out_specs=[
            pl.BlockSpec((1, 1, n), lambda i: (i, 0, 0)),
            pl.BlockSpec((1, 1, d), lambda i: (i, 0, 0)),
        ],
        out_shape=[
            jax.ShapeDtypeStruct((b, 1, n), jnp.float32),
            jax.ShapeDtypeStruct((b, 1, d), jnp.float32),
        ],
        compiler_params=pltpu.CompilerParams(
            dimension_semantics=("parallel",),
        ),
    )(x, Wa1, ba1r, wa2t, ba2r)
    weights = weights.reshape(b, n)
    embs = embs.reshape(b, d)

    projection = pl.pallas_call(
        _proj_body,
        in_specs=[
            pl.BlockSpec((b, d), lambda: (0, 0)),
            pl.BlockSpec((d, d), lambda: (0, 0)),
            pl.BlockSpec((1, d), lambda: (0, 0)),
            pl.BlockSpec((d, d), lambda: (0, 0)),
            pl.BlockSpec((1, d), lambda: (0, 0)),
        ],
        out_specs=pl.BlockSpec((b, d), lambda: (0, 0)),
        out_shape=jax.ShapeDtypeStruct((b, d), jnp.float32),
    )(embs, Wp1, bp1r, Wp2, bp2r)

    return projection, weights


# speedup vs baseline: 1.4069x; 1.4069x over previous
import functools

import jax
import jax.numpy as jnp
from jax import lax
from jax.experimental import pallas as pl
from jax.experimental.pallas import tpu as pltpu

_TOPK_PERCENT = 0.3


def _select_weights(w_row, k):
    n = w_row.shape[1]
    wi = lax.bitcast_convert_type(w_row, jnp.int32)
    wb = jnp.broadcast_to(wi, (8, n))
    s1 = lax.broadcasted_iota(jnp.int32, (8, 1), 0) + 1

    def bs_body(_, lohi):
        lo, hi = lohi
        step = jnp.maximum((hi - lo) // 8, 1)
        t = jnp.minimum(lo + s1 * step, hi)
        cnt = jnp.sum((wb >= t).astype(jnp.int32), axis=1, keepdims=True)
        ge = cnt >= k
        lo = jnp.max(jnp.where(ge, t, lo), axis=0, keepdims=True)
        hi = jnp.min(jnp.where(ge, hi, t), axis=0, keepdims=True)
        return lo, hi

    lo, _ = lax.fori_loop(
        0, 12, bs_body,
        (jnp.zeros((1, 1), jnp.int32), jnp.full((1, 1), 0x3F800001, jnp.int32)),
    )
    t = lo

    gt = wi > t
    eq = wi == t
    n_gt = jnp.sum(gt.astype(jnp.int32), axis=1, keepdims=True)
    extra = k - n_gt

    idx = lax.broadcasted_iota(jnp.int32, (1, n), 1)
    eqb = jnp.broadcast_to(eq, (8, n))

    def bs2_body(_, lohi):
        lo2, hi2 = lohi
        t2 = lo2 + (s1 * (hi2 - lo2)) // 8
        cnt = jnp.sum(
            (eqb & (idx < t2)).astype(jnp.int32), axis=1, keepdims=True
        )
        ge = cnt >= extra
        hi2 = jnp.min(jnp.where(ge, t2, hi2), axis=0, keepdims=True)
        lo2 = jnp.max(jnp.where(ge, lo2, t2), axis=0, keepdims=True)
        return lo2, hi2

    _, m = lax.fori_loop(
        0, 5, bs2_body,
        (jnp.zeros((1, 1), jnp.int32), jnp.full((1, 1), n, jnp.int32)),
    )

    sel = gt | (eq & (idx < m))
    return jnp.where(sel, w_row, 0.0)


def _main_body(x_ref, wa1_ref, ba1_ref, wa2t_ref, ba2_ref, w_ref, emb_ref,
               *, k, nc=1024):
    n = x_ref.shape[1]

    for c in range(n // nc):
        xb = x_ref[0, pl.ds(c * nc, nc), :]
        h = jax.nn.gelu(
            jnp.dot(xb, wa1_ref[...], preferred_element_type=jnp.float32)
            + ba1_ref[...]
        )
        z = (
            lax.dot_general(
                wa2t_ref[...], h,
                dimension_numbers=(((1,), (1,)), ((), ())),
                preferred_element_type=jnp.float32,
            )
            + ba2_ref[...]
        )
        w_ref[0, :, pl.ds(c * nc, nc)] = jax.nn.sigmoid(z)

    w_row = w_ref[0]
    wm_row = _select_weights(w_row, k)

    emb_ref[0] = jnp.dot(
        wm_row, x_ref[0], preferred_element_type=jnp.float32
    ) * (1.0 / k)


def _proj_body(emb_ref, wp1_ref, bp1_ref, wp2_ref, bp2_ref, out_ref):
    h = jax.nn.gelu(
        jnp.dot(emb_ref[...], wp1_ref[...], preferred_element_type=jnp.float32)
        + bp1_ref[...]
    )
    out_ref[...] = (
        jnp.dot(h, wp2_ref[...], preferred_element_type=jnp.float32) + bp2_ref[...]
    )


def kernel(x, Wa1, ba1, Wa2, ba2, Wp1, bp1, Wp2, bp2):
    b, n, d = x.shape
    hdim = Wa1.shape[1]
    k = max(1, int(n * _TOPK_PERCENT))

    ba1r = ba1.reshape(1, hdim)
    wa2t = Wa2.reshape(1, hdim)
    ba2r = ba2.reshape(1, 1)
    bp1r = bp1.reshape(1, d)
    bp2r = bp2.reshape(1, d)

    weights, embs = pl.pallas_call(
        functools.partial(_main_body, k=k),
        grid=(b,),
        in_specs=[
            pl.BlockSpec((1, n, d), lambda i: (i, 0, 0)),
            pl.BlockSpec((d, hdim), lambda i: (0, 0)),
            pl.BlockSpec((1, hdim), lambda i: (0, 0)),
            pl.BlockSpec((1, hdim), lambda i: (0, 0)),
            pl.BlockSpec((1, 1), lambda i: (0, 0)),
        ],
        out_specs=[
            pl.BlockSpec((1, 1, n), lambda i: (i, 0, 0)),
            pl.BlockSpec((1, 1, d), lambda i: (i, 0, 0)),
        ],
        out_shape=[
            jax.ShapeDtypeStruct((b, 1, n), jnp.float32),
            jax.ShapeDtypeStruct((b, 1, d), jnp.float32),
        ],
        compiler_params=pltpu.CompilerParams(
            dimension_semantics=("parallel",),
        ),
    )(x, Wa1, ba1r, wa2t, ba2r)
    weights = weights.reshape(b, n)
    embs = embs.reshape(b, d)

    projection = pl.pallas_call(
        _proj_body,
        in_specs=[
            pl.BlockSpec((b, d), lambda: (0, 0)),
            pl.BlockSpec((d, d), lambda: (0, 0)),
            pl.BlockSpec((1, d), lambda: (0, 0)),
            pl.BlockSpec((d, d), lambda: (0, 0)),
            pl.BlockSpec((1, d), lambda: (0, 0)),
        ],
        out_specs=pl.BlockSpec((b, d), lambda: (0, 0)),
        out_shape=jax.ShapeDtypeStruct((b, d), jnp.float32),
    )(embs, Wp1, bp1r, Wp2, bp2r)

    return projection, weights


# speedup vs baseline: 1.6006x; 1.1377x over previous
import functools

import jax
import jax.numpy as jnp
from jax import lax
from jax.experimental import pallas as pl
from jax.experimental.pallas import tpu as pltpu

_TOPK_PERCENT = 0.3


def _select_weights(w_row, k):
    n = w_row.shape[1]
    wi = lax.bitcast_convert_type(w_row, jnp.int32)
    wb = jnp.broadcast_to(wi, (8, n))
    s1 = lax.broadcasted_iota(jnp.int32, (8, 1), 0) + 1

    def bs_body(_, lohi):
        lo, hi = lohi
        step = jnp.maximum((hi - lo) // 8, 1)
        t = jnp.minimum(lo + s1 * step, hi)
        cnt = jnp.sum((wb >= t).astype(jnp.int32), axis=1, keepdims=True)
        ge = cnt >= k
        lo = jnp.max(jnp.where(ge, t, lo), axis=0, keepdims=True)
        hi = jnp.min(jnp.where(ge, hi, t), axis=0, keepdims=True)
        return lo, hi

    lo, _ = lax.fori_loop(
        0, 12, bs_body,
        (jnp.zeros((1, 1), jnp.int32), jnp.full((1, 1), 0x3F800001, jnp.int32)),
    )
    t = lo

    gt = wi > t
    eq = wi == t
    n_gt = jnp.sum(gt.astype(jnp.int32), axis=1, keepdims=True)
    n_eq = jnp.sum(eq.astype(jnp.int32), axis=1, keepdims=True)
    extra = k - n_gt

    idx = lax.broadcasted_iota(jnp.int32, (1, n), 1)

    def tie_search():
        eqb = jnp.broadcast_to(eq, (8, n))

        def bs2_body(_, lohi):
            lo2, hi2 = lohi
            t2 = lo2 + (s1 * (hi2 - lo2)) // 8
            cnt = jnp.sum(
                (eqb & (idx < t2)).astype(jnp.int32), axis=1, keepdims=True
            )
            ge = cnt >= extra
            hi2n = jnp.min(jnp.where(ge, t2, hi2), axis=0, keepdims=True)
            lo2n = jnp.max(jnp.where(ge, lo2, t2), axis=0, keepdims=True)
            return lo2n, hi2n

        _, m = lax.fori_loop(
            0, 5, bs2_body,
            (jnp.zeros((1, 1), jnp.int32), jnp.full((1, 1), n, jnp.int32)),
        )
        return m

    m = lax.cond(
        n_eq[0, 0] == extra[0, 0],
        lambda: jnp.full((1, 1), n, jnp.int32),
        tie_search,
    )

    sel = gt | (eq & (idx < m))
    return jnp.where(sel, w_row, 0.0)


def _main_body(x_ref, wa1_ref, ba1_ref, wa2t_ref, ba2_ref, w_ref, emb_ref,
               *, k, nc=2048):
    n = x_ref.shape[1]

    for c in range(n // nc):
        xb = x_ref[0, pl.ds(c * nc, nc), :]
        h = jax.nn.gelu(
            jnp.dot(xb, wa1_ref[...], preferred_element_type=jnp.float32)
            + ba1_ref[...]
        )
        z = (
            lax.dot_general(
                wa2t_ref[...], h,
                dimension_numbers=(((1,), (1,)), ((), ())),
                preferred_element_type=jnp.float32,
            )
            + ba2_ref[...]
        )
        w_ref[0, :, pl.ds(c * nc, nc)] = jax.nn.sigmoid(z)

    w_row = w_ref[0]
    wm_row = _select_weights(w_row, k)

    emb_ref[0] = jnp.dot(
        wm_row, x_ref[0], preferred_element_type=jnp.float32
    ) * (1.0 / k)


def _proj_body(emb_ref, wp1_ref, bp1_ref, wp2_ref, bp2_ref, out_ref):
    h = jax.nn.gelu(
        jnp.dot(emb_ref[...], wp1_ref[...], preferred_element_type=jnp.float32)
        + bp1_ref[...]
    )
    out_ref[...] = (
        jnp.dot(h, wp2_ref[...], preferred_element_type=jnp.float32) + bp2_ref[...]
    )


def kernel(x, Wa1, ba1, Wa2, ba2, Wp1, bp1, Wp2, bp2):
    b, n, d = x.shape
    hdim = Wa1.shape[1]
    k = max(1, int(n * _TOPK_PERCENT))

    ba1r = ba1.reshape(1, hdim)
    wa2t = Wa2.reshape(1, hdim)
    ba2r = ba2.reshape(1, 1)
    bp1r = bp1.reshape(1, d)
    bp2r = bp2.reshape(1, d)

    weights, embs = pl.pallas_call(
        functools.partial(_main_body, k=k),
        grid=(b,),
        in_specs=[
            pl.BlockSpec((1, n, d), lambda i: (i, 0, 0)),
            pl.BlockSpec((d, hdim), lambda i: (0, 0)),
            pl.BlockSpec((1, hdim), lambda i: (0, 0)),
            pl.BlockSpec((1, hdim), lambda i: (0, 0)),
            pl.BlockSpec((1, 1), lambda i: (0, 0)),
        ],
        out_specs=[
            pl.BlockSpec((1, 1, n), lambda i: (i, 0, 0)),
            pl.BlockSpec((1, 1, d), lambda i: (i, 0, 0)),
        ],
        out_shape=[
            jax.ShapeDtypeStruct((b, 1, n), jnp.float32),
            jax.ShapeDtypeStruct((b, 1, d), jnp.float32),
        ],
        compiler_params=pltpu.CompilerParams(
            dimension_semantics=("parallel",),
        ),
    )(x, Wa1, ba1r, wa2t, ba2r)
    weights = weights.reshape(b, n)
    embs = embs.reshape(b, d)

    projection = pl.pallas_call(
        _proj_body,
        in_specs=[
            pl.BlockSpec((b, d), lambda: (0, 0)),
            pl.BlockSpec((d, d), lambda: (0, 0)),
            pl.BlockSpec((1, d), lambda: (0, 0)),
            pl.BlockSpec((d, d), lambda: (0, 0)),
            pl.BlockSpec((1, d), lambda: (0, 0)),
        ],
        out_specs=pl.BlockSpec((b, d), lambda: (0, 0)),
        out_shape=jax.ShapeDtypeStruct((b, d), jnp.float32),
    )(embs, Wp1, bp1r, Wp2, bp2r)

    return projection, weights


# speedup vs baseline: 2.1497x; 1.3431x over previous
import functools

import jax
import jax.numpy as jnp
from jax import lax
from jax.experimental import pallas as pl
from jax.experimental.pallas import tpu as pltpu

_TOPK_PERCENT = 0.3


def _select_weights(w_row, k):
    n = w_row.shape[1]
    wi = lax.bitcast_convert_type(w_row, jnp.int32)
    wb = jnp.broadcast_to(wi, (8, n))
    s1 = lax.broadcasted_iota(jnp.int32, (8, 1), 0) + 1

    def bs_body(_, lohi):
        lo, hi = lohi
        step = jnp.maximum((hi - lo) // 8, 1)
        t = jnp.minimum(lo + s1 * step, hi)
        cnt = jnp.sum((wb >= t).astype(jnp.int32), axis=1, keepdims=True)
        ge = cnt >= k
        lo = jnp.max(jnp.where(ge, t, lo), axis=0, keepdims=True)
        hi = jnp.min(jnp.where(ge, hi, t), axis=0, keepdims=True)
        return lo, hi

    lo, _ = lax.fori_loop(
        0, 12, bs_body,
        (jnp.zeros((1, 1), jnp.int32), jnp.full((1, 1), 0x3F800001, jnp.int32)),
    )
    t = lo

    gt = wi > t
    eq = wi == t
    n_gt = jnp.sum(gt.astype(jnp.int32), axis=1, keepdims=True)
    n_eq = jnp.sum(eq.astype(jnp.int32), axis=1, keepdims=True)
    extra = k - n_gt

    idx = lax.broadcasted_iota(jnp.int32, (1, n), 1)

    def tie_search():
        eqb = jnp.broadcast_to(eq, (8, n))

        def bs2_body(_, lohi):
            lo2, hi2 = lohi
            t2 = lo2 + (s1 * (hi2 - lo2)) // 8
            cnt = jnp.sum(
                (eqb & (idx < t2)).astype(jnp.int32), axis=1, keepdims=True
            )
            ge = cnt >= extra
            hi2n = jnp.min(jnp.where(ge, t2, hi2), axis=0, keepdims=True)
            lo2n = jnp.max(jnp.where(ge, lo2, t2), axis=0, keepdims=True)
            return lo2n, hi2n

        _, m = lax.fori_loop(
            0, 5, bs2_body,
            (jnp.zeros((1, 1), jnp.int32), jnp.full((1, 1), n, jnp.int32)),
        )
        return m

    m = lax.cond(
        n_eq[0, 0] == extra[0, 0],
        lambda: jnp.full((1, 1), n, jnp.int32),
        tie_search,
    )

    sel = gt | (eq & (idx < m))
    return jnp.where(sel, w_row, 0.0)


def _main_body(x_ref, wa1t_ref, ba1c_ref, wa2t_ref, ba2_ref, w_ref, emb_ref,
               *, k, nc=2048):
    n = x_ref.shape[1]

    for c in range(n // nc):
        xb = x_ref[0, pl.ds(c * nc, nc), :]
        ht = jax.nn.gelu(
            lax.dot_general(
                wa1t_ref[...], xb,
                dimension_numbers=(((1,), (1,)), ((), ())),
                preferred_element_type=jnp.float32,
            )
            + ba1c_ref[...]
        )
        z = (
            jnp.dot(wa2t_ref[...], ht, preferred_element_type=jnp.float32)
            + ba2_ref[...]
        )
        w_ref[0, :, pl.ds(c * nc, nc)] = jax.nn.sigmoid(z)

    w_row = w_ref[0]
    wm_row = _select_weights(w_row, k)

    emb_ref[0] = jnp.dot(
        wm_row, x_ref[0], preferred_element_type=jnp.float32
    ) * (1.0 / k)


def _proj_body(emb_ref, wp1_ref, bp1_ref, wp2_ref, bp2_ref, out_ref):
    h = jax.nn.gelu(
        jnp.dot(emb_ref[...], wp1_ref[...], preferred_element_type=jnp.float32)
        + bp1_ref[...]
    )
    out_ref[...] = (
        jnp.dot(h, wp2_ref[...], preferred_element_type=jnp.float32) + bp2_ref[...]
    )


def kernel(x, Wa1, ba1, Wa2, ba2, Wp1, bp1, Wp2, bp2):
    b, n, d = x.shape
    hdim = Wa1.shape[1]
    k = max(1, int(n * _TOPK_PERCENT))

    wa1t = Wa1.T
    ba1c = ba1.reshape(hdim, 1)
    wa2t = Wa2.reshape(1, hdim)
    ba2r = ba2.reshape(1, 1)
    bp1r = bp1.reshape(1, d)
    bp2r = bp2.reshape(1, d)

    weights, embs = pl.pallas_call(
        functools.partial(_main_body, k=k),
        grid=(b,),
        in_specs=[
            pl.BlockSpec((1, n, d), lambda i: (i, 0, 0)),
            pl.BlockSpec((hdim, d), lambda i: (0, 0)),
            pl.BlockSpec((hdim, 1), lambda i: (0, 0)),
            pl.BlockSpec((1, hdim), lambda i: (0, 0)),
            pl.BlockSpec((1, 1), lambda i: (0, 0)),
        ],
        out_specs=[
            pl.BlockSpec((1, 1, n), lambda i: (i, 0, 0)),
            pl.BlockSpec((1, 1, d), lambda i: (i, 0, 0)),
        ],
        out_shape=[
            jax.ShapeDtypeStruct((b, 1, n), jnp.float32),
            jax.ShapeDtypeStruct((b, 1, d), jnp.float32),
        ],
        compiler_params=pltpu.CompilerParams(
            dimension_semantics=("parallel",),
        ),
    )(x, wa1t, ba1c, wa2t, ba2r)
    weights = weights.reshape(b, n)
    embs = embs.reshape(b, d)

    projection = pl.pallas_call(
        _proj_body,
        in_specs=[
            pl.BlockSpec((b, d), lambda: (0, 0)),
            pl.BlockSpec((d, d), lambda: (0, 0)),
            pl.BlockSpec((1, d), lambda: (0, 0)),
            pl.BlockSpec((d, d), lambda: (0, 0)),
            pl.BlockSpec((1, d), lambda: (0, 0)),
        ],
        out_specs=pl.BlockSpec((b, d), lambda: (0, 0)),
        out_shape=jax.ShapeDtypeStruct((b, d), jnp.float32),
    )(embs, Wp1, bp1r, Wp2, bp2r)

    return projection, weights


# speedup vs baseline: 2.2188x; 1.0321x over previous
import functools

import jax
import jax.numpy as jnp
from jax import lax
from jax.experimental import pallas as pl
from jax.experimental.pallas import tpu as pltpu

_TOPK_PERCENT = 0.3


def _select_weights(w_row, k):
    n = w_row.shape[1]
    wi = lax.bitcast_convert_type(w_row, jnp.int32)
    wb = jnp.broadcast_to(wi, (8, n))
    s1 = lax.broadcasted_iota(jnp.int32, (8, 1), 0) + 1

    def bs_body(_, lohi):
        lo, hi = lohi
        step = jnp.maximum((hi - lo) // 8, 1)
        t = jnp.minimum(lo + s1 * step, hi)
        cnt = jnp.sum((wb >= t).astype(jnp.int32), axis=1, keepdims=True)
        ge = cnt >= k
        lo = jnp.max(jnp.where(ge, t, lo), axis=0, keepdims=True)
        hi = jnp.min(jnp.where(ge, hi, t), axis=0, keepdims=True)
        return lo, hi

    lo, _ = lax.fori_loop(
        0, 12, bs_body,
        (jnp.zeros((1, 1), jnp.int32), jnp.full((1, 1), 0x3F800001, jnp.int32)),
    )
    t = lo

    gt = wi > t
    eq = wi == t
    n_gt = jnp.sum(gt.astype(jnp.int32), axis=1, keepdims=True)
    n_eq = jnp.sum(eq.astype(jnp.int32), axis=1, keepdims=True)
    extra = k - n_gt

    idx = lax.broadcasted_iota(jnp.int32, (1, n), 1)

    def tie_search():
        eqb = jnp.broadcast_to(eq, (8, n))

        def bs2_body(_, lohi):
            lo2, hi2 = lohi
            t2 = lo2 + (s1 * (hi2 - lo2)) // 8
            cnt = jnp.sum(
                (eqb & (idx < t2)).astype(jnp.int32), axis=1, keepdims=True
            )
            ge = cnt >= extra
            hi2n = jnp.min(jnp.where(ge, t2, hi2), axis=0, keepdims=True)
            lo2n = jnp.max(jnp.where(ge, lo2, t2), axis=0, keepdims=True)
            return lo2n, hi2n

        _, m = lax.fori_loop(
            0, 5, bs2_body,
            (jnp.zeros((1, 1), jnp.int32), jnp.full((1, 1), n, jnp.int32)),
        )
        return m

    m = lax.cond(
        n_eq[0, 0] == extra[0, 0],
        lambda: jnp.full((1, 1), n, jnp.int32),
        tie_search,
    )

    sel = gt | (eq & (idx < m))
    return jnp.where(sel, w_row, 0.0)


def _main_body(x_ref, wa1t_ref, ba1c_ref, wa2t_ref, ba2_ref,
               wp1_ref, bp1_ref, wp2_ref, bp2_ref,
               w_ref, proj_ref, emb_scratch, *, k, nc=2048):
    n = x_ref.shape[1]

    for c in range(n // nc):
        xb = x_ref[0, pl.ds(c * nc, nc), :]
        ht = jax.nn.gelu(
            lax.dot_general(
                wa1t_ref[...], xb,
                dimension_numbers=(((1,), (1,)), ((), ())),
                preferred_element_type=jnp.float32,
            )
            + ba1c_ref[...]
        )
        z = (
            jnp.dot(wa2t_ref[...], ht, preferred_element_type=jnp.float32)
            + ba2_ref[...]
        )
        w_ref[0, :, pl.ds(c * nc, nc)] = jax.nn.sigmoid(z)

    w_row = w_ref[0]
    wm_row = _select_weights(w_row, k)

    i = pl.program_id(0)
    nb = pl.num_programs(0)
    emb_scratch[pl.ds(i, 1), :] = jnp.dot(
        wm_row, x_ref[0], preferred_element_type=jnp.float32
    ) * (1.0 / k)

    @pl.when(i == nb - 1)
    def _():
        embs = emb_scratch[...]
        h = jax.nn.gelu(
            jnp.dot(embs, wp1_ref[...], preferred_element_type=jnp.float32)
            + bp1_ref[...]
        )
        proj_ref[...] = (
            jnp.dot(h, wp2_ref[...], preferred_element_type=jnp.float32)
            + bp2_ref[...]
        )


def kernel(x, Wa1, ba1, Wa2, ba2, Wp1, bp1, Wp2, bp2):
    b, n, d = x.shape
    hdim = Wa1.shape[1]
    k = max(1, int(n * _TOPK_PERCENT))

    wa1t = Wa1.T
    ba1c = ba1.reshape(hdim, 1)
    wa2t = Wa2.reshape(1, hdim)
    ba2r = ba2.reshape(1, 1)
    bp1r = bp1.reshape(1, d)
    bp2r = bp2.reshape(1, d)

    weights, projection = pl.pallas_call(
        functools.partial(_main_body, k=k),
        grid=(b,),
        in_specs=[
            pl.BlockSpec((1, n, d), lambda i: (i, 0, 0)),
            pl.BlockSpec((hdim, d), lambda i: (0, 0)),
            pl.BlockSpec((hdim, 1), lambda i: (0, 0)),
            pl.BlockSpec((1, hdim), lambda i: (0, 0)),
            pl.BlockSpec((1, 1), lambda i: (0, 0)),
            pl.BlockSpec((d, d), lambda i: (0, 0)),
            pl.BlockSpec((1, d), lambda i: (0, 0)),
            pl.BlockSpec((d, d), lambda i: (0, 0)),
            pl.BlockSpec((1, d), lambda i: (0, 0)),
        ],
        out_specs=[
            pl.BlockSpec((1, 1, n), lambda i: (i, 0, 0)),
            pl.BlockSpec((b, d), lambda i: (0, 0)),
        ],
        out_shape=[
            jax.ShapeDtypeStruct((b, 1, n), jnp.float32),
            jax.ShapeDtypeStruct((b, d), jnp.float32),
        ],
        scratch_shapes=[pltpu.VMEM((b, d), jnp.float32)],
        compiler_params=pltpu.CompilerParams(
            dimension_semantics=("arbitrary",),
        ),
    )(x, wa1t, ba1c, wa2t, ba2r, Wp1, bp1r, Wp2, bp2r)
    weights = weights.reshape(b, n)

    return projection, weights
